# baseline (speedup 1.0000x reference)
import functools

import jax
import jax.numpy as jnp
from jax.experimental import pallas as pl
from jax.experimental.pallas import tpu as pltpu

D = 128
H = 256
THRESH = 0.99
SLOPE = 0.2
NEG = -1e9


def _leaky(x):
    return jnp.where(x >= 0, x, SLOPE * x)


def _proj_body(x_ref, w_ref, aq_ref, ak_ref, h_ref, q_ref, k_ref):
    h = jax.lax.dot(x_ref[...], w_ref[...], preferred_element_type=jnp.float32)
    h_ref[...] = h
    q_ref[...] = jax.lax.dot(h, aq_ref[...], preferred_element_type=jnp.float32)
    k_ref[...] = jax.lax.dot(h, ak_ref[...], preferred_element_type=jnp.float32)


def _proj(x, w, aq, ak):
    n = x.shape[0]
    ti = min(512, n)
    grid = (n // ti,)
    return pl.pallas_call(
        _proj_body,
        grid=grid,
        in_specs=[
            pl.BlockSpec((ti, D), lambda i: (i, 0)),
            pl.BlockSpec((D, D), lambda i: (0, 0)),
            pl.BlockSpec((D, 1), lambda i: (0, 0)),
            pl.BlockSpec((D, 1), lambda i: (0, 0)),
        ],
        out_specs=[
            pl.BlockSpec((ti, D), lambda i: (i, 0)),
            pl.BlockSpec((ti, 1), lambda i: (i, 0)),
            pl.BlockSpec((ti, 1), lambda i: (i, 0)),
        ],
        out_shape=[
            jax.ShapeDtypeStruct((n, D), jnp.float32),
            jax.ShapeDtypeStruct((n, 1), jnp.float32),
            jax.ShapeDtypeStruct((n, 1), jnp.float32),
        ],
    )(x, w, aq.reshape(D, 1), ak.reshape(D, 1))


def _projvec_body(x_ref, w_ref, av_ref, k_ref):
    wv = jax.lax.dot(w_ref[...], av_ref[...], preferred_element_type=jnp.float32)
    k_ref[...] = jax.lax.dot(x_ref[...], wv, preferred_element_type=jnp.float32)


def _projvec(x, w, av):
    n = x.shape[0]
    ti = min(512, n)
    return pl.pallas_call(
        _projvec_body,
        grid=(n // ti,),
        in_specs=[
            pl.BlockSpec((ti, D), lambda i: (i, 0)),
            pl.BlockSpec((D, D), lambda i: (0, 0)),
            pl.BlockSpec((D, 1), lambda i: (0, 0)),
        ],
        out_specs=pl.BlockSpec((ti, 1), lambda i: (i, 0)),
        out_shape=jax.ShapeDtypeStruct((n, 1), jnp.float32),
    )(x, w, av.reshape(D, 1))


def _row_flash_body(nj, has_extra, has_res, leaky_out, *refs):
    q_ref, k_ref, a_ref, h_ref = refs[:4]
    idx = 4
    e_ref = r_ref = None
    if has_extra:
        e_ref = refs[idx]
        idx += 1
    if has_res:
        r_ref = refs[idx]
        idx += 1
    o_ref, m_ref, l_ref = refs[idx], refs[idx + 1], refs[idx + 2]
    j = pl.program_id(1)

    @pl.when(j == 0)
    def _():
        m_ref[...] = jnp.full(m_ref.shape, -jnp.inf, jnp.float32)
        l_ref[...] = jnp.zeros(l_ref.shape, jnp.float32)
        o_ref[...] = jnp.zeros(o_ref.shape, jnp.float32)

    s = q_ref[...] + k_ref[...]
    s = jnp.where(s >= 0, s, SLOPE * s)
    s = jnp.where(a_ref[...] > THRESH, s, NEG)
    m_old = m_ref[...]
    m_new = jnp.maximum(m_old, jnp.max(s, axis=1, keepdims=True))
    alpha = jnp.exp(m_old - m_new)
    e = jnp.exp(s - m_new)
    l_ref[...] = l_ref[...] * alpha + jnp.sum(e, axis=1, keepdims=True)
    o_ref[...] = o_ref[...] * alpha + jax.lax.dot(
        e, h_ref[...], preferred_element_type=jnp.float32)
    m_ref[...] = m_new

    @pl.when(j == nj - 1)
    def _():
        r = o_ref[...] / (l_ref[...] + 1e-9)
        if has_extra:
            r = r + e_ref[...]
        if leaky_out:
            r = jnp.where(r >= 0, r, SLOPE * r)
        if has_res:
            r = r + r_ref[...]
        o_ref[...] = r


def _row_flash(q, kt, adj, h, extra=None, residual=None, leaky_out=False):
    nr, nc = adj.shape
    ti = min(256, nr)
    tj = min(512, nc)
    gi, gj = nr // ti, nc // tj
    inputs = [q, kt, adj, h]
    specs = [
        pl.BlockSpec((ti, 1), lambda i, j: (i, 0)),
        pl.BlockSpec((1, tj), lambda i, j: (0, j)),
        pl.BlockSpec((ti, tj), lambda i, j: (i, j)),
        pl.BlockSpec((tj, D), lambda i, j: (j, 0)),
    ]
    if extra is not None:
        inputs.append(extra)
        specs.append(pl.BlockSpec((ti, D), lambda i, j: (i, 0)))
    if residual is not None:
        inputs.append(residual)
        specs.append(pl.BlockSpec((ti, D), lambda i, j: (i, 0)))
    body = functools.partial(
        _row_flash_body, gj, extra is not None, residual is not None, leaky_out)
    return pl.pallas_call(
        body,
        grid=(gi, gj),
        in_specs=specs,
        out_specs=pl.BlockSpec((ti, D), lambda i, j: (i, 0)),
        out_shape=jax.ShapeDtypeStruct((nr, D), jnp.float32),
        scratch_shapes=[
            pltpu.VMEM((ti, 1), jnp.float32),
            pltpu.VMEM((ti, 1), jnp.float32),
        ],
    )(*inputs)


def _col_flash_body(ns_, q_ref, k_ref, a_ref, h_ref, o_ref, m_ref, l_ref):
    s_id = pl.program_id(1)

    @pl.when(s_id == 0)
    def _():
        m_ref[...] = jnp.full(m_ref.shape, -jnp.inf, jnp.float32)
        l_ref[...] = jnp.zeros(l_ref.shape, jnp.float32)
        o_ref[...] = jnp.zeros(o_ref.shape, jnp.float32)

    sc = q_ref[...] + k_ref[...]
    sc = jnp.where(sc >= 0, sc, SLOPE * sc)
    sc = jnp.where(a_ref[...] > THRESH, sc, NEG)
    m_old = m_ref[...]
    m_new = jnp.maximum(m_old, jnp.max(sc, axis=0, keepdims=True))
    alpha = jnp.exp(m_old - m_new)
    e = jnp.exp(sc - m_new)
    l_new = l_ref[...] * alpha + jnp.sum(e, axis=0, keepdims=True)
    l_ref[...] = l_new
    m_ref[...] = m_new
    contrib = jax.lax.dot_general(
        e, h_ref[...], (((0,), (0,)), ((), ())),
        preferred_element_type=jnp.float32)
    acc = o_ref[...] * alpha.T + contrib

    @pl.when(s_id < ns_ - 1)
    def _():
        o_ref[...] = acc

    @pl.when(s_id == ns_ - 1)
    def _():
        o_ref[...] = acc / (l_new.T + 1e-9)


def _col_flash(q, kt, adj, hs):
    ns, nt = adj.shape
    ts = min(512, ns)
    tt = min(256, nt)
    gs, gt = ns // ts, nt // tt
    body = functools.partial(_col_flash_body, gs)
    return pl.pallas_call(
        body,
        grid=(gt, gs),
        in_specs=[
            pl.BlockSpec((ts, 1), lambda t, s: (s, 0)),
            pl.BlockSpec((1, tt), lambda t, s: (0, t)),
            pl.BlockSpec((ts, tt), lambda t, s: (s, t)),
            pl.BlockSpec((ts, D), lambda t, s: (s, 0)),
        ],
        out_specs=pl.BlockSpec((tt, D), lambda t, s: (t, 0)),
        out_shape=jax.ShapeDtypeStruct((nt, D), jnp.float32),
        scratch_shapes=[
            pltpu.VMEM((1, tt), jnp.float32),
            pltpu.VMEM((1, tt), jnp.float32),
        ],
    )(q, kt, adj, hs)


def _dual_body(gi, gj, tj, has_extra_s, leaky_s, leaky_t,
               *refs):
    q_ref, k_ref, a_ref, hs_ref, ht_ref = refs[:5]
    idx = 5
    es_ref = None
    if has_extra_s:
        es_ref = refs[idx]
        idx += 1
    os_ref, ot_ref = refs[idx], refs[idx + 1]
    mr_ref, lr_ref, mc_ref, lc_ref = refs[idx + 2:idx + 6]
    i = pl.program_id(0)
    j = pl.program_id(1)

    @pl.when(j == 0)
    def _():
        mr_ref[...] = jnp.full(mr_ref.shape, -jnp.inf, jnp.float32)
        lr_ref[...] = jnp.zeros(lr_ref.shape, jnp.float32)
        os_ref[...] = jnp.zeros(os_ref.shape, jnp.float32)

    cds = pl.ds(j * tj, tj)

    @pl.when(i == 0)
    def _():
        mc_ref[:, cds] = jnp.full((1, tj), -jnp.inf, jnp.float32)
        lc_ref[:, cds] = jnp.zeros((1, tj), jnp.float32)
        ot_ref[pl.ds(j * tj, tj), :] = jnp.zeros((tj, D), jnp.float32)

    s = q_ref[...] + k_ref[...]
    s = jnp.where(s >= 0, s, SLOPE * s)
    s = jnp.where(a_ref[...] > THRESH, s, NEG)

    m_old = mr_ref[...]
    m_new = jnp.maximum(m_old, jnp.max(s, axis=1, keepdims=True))
    alpha = jnp.exp(m_old - m_new)
    e = jnp.exp(s - m_new)
    lr_new = lr_ref[...] * alpha + jnp.sum(e, axis=1, keepdims=True)
    lr_ref[...] = lr_new
    mr_ref[...] = m_new
    acc_s = os_ref[...] * alpha + jax.lax.dot(
        e, ht_ref[...], preferred_element_type=jnp.float32)

    @pl.when(j < gj - 1)
    def _():
        os_ref[...] = acc_s

    @pl.when(j == gj - 1)
    def _():
        r = acc_s / (lr_new + 1e-9)
        if has_extra_s:
            r = r + es_ref[...]
        if leaky_s:
            r = jnp.where(r >= 0, r, SLOPE * r)
        os_ref[...] = r

    mc_old = mc_ref[:, cds]
    mc_new = jnp.maximum(mc_old, jnp.max(s, axis=0, keepdims=True))
    alpha_c = jnp.exp(mc_old - mc_new)
    e_c = jnp.exp(s - mc_new)
    lc_new = lc_ref[:, cds] * alpha_c + jnp.sum(e_c, axis=0, keepdims=True)
    lc_ref[:, cds] = lc_new
    mc_ref[:, cds] = mc_new
    contrib = jax.lax.dot_general(
        e_c, hs_ref[...], (((0,), (0,)), ((), ())),
        preferred_element_type=jnp.float32)
    acc_t = ot_ref[cds, :] * alpha_c.T + contrib

    @pl.when(i < gi - 1)
    def _():
        ot_ref[cds, :] = acc_t

    @pl.when(i == gi - 1)
    def _():
        r = acc_t / (lc_new.T + 1e-9)
        if leaky_t:
            r = jnp.where(r >= 0, r, SLOPE * r)
        ot_ref[cds, :] = r


def _dual_flash(q, kt, adj, hs, ht, extra_s=None, leaky_s=False, leaky_t=False):
    ns, nt = adj.shape
    ti = min(256, ns)
    tj = min(512, nt)
    gi, gj = ns // ti, nt // tj
    inputs = [q, kt, adj, hs, ht]
    specs = [
        pl.BlockSpec((ti, 1), lambda i, j: (i, 0)),
        pl.BlockSpec((1, tj), lambda i, j: (0, j)),
        pl.BlockSpec((ti, tj), lambda i, j: (i, j)),
        pl.BlockSpec((ti, D), lambda i, j: (i, 0)),
        pl.BlockSpec((tj, D), lambda i, j: (j, 0)),
    ]
    if extra_s is not None:
        inputs.append(extra_s)
        specs.append(pl.BlockSpec((ti, D), lambda i, j: (i, 0)))
    body = functools.partial(
        _dual_body, gi, gj, tj, extra_s is not None, leaky_s, leaky_t)
    return pl.pallas_call(
        body,
        grid=(gi, gj),
        in_specs=specs,
        out_specs=[
            pl.BlockSpec((ti, D), lambda i, j: (i, 0)),
            pl.BlockSpec((nt, D), lambda i, j: (0, 0)),
        ],
        out_shape=[
            jax.ShapeDtypeStruct((ns, D), jnp.float32),
            jax.ShapeDtypeStruct((nt, D), jnp.float32),
        ],
        scratch_shapes=[
            pltpu.VMEM((ti, 1), jnp.float32),
            pltpu.VMEM((ti, 1), jnp.float32),
            pltpu.VMEM((1, nt), jnp.float32),
            pltpu.VMEM((1, nt), jnp.float32),
        ],
    )(*inputs)


def _dec_body(x_ref, wi_ref, bi_ref, wm_ref, bm_ref, wo_ref, bo_ref, o_ref):
    z = jax.lax.dot(x_ref[...], wi_ref[...], preferred_element_type=jnp.float32)
    z = jax.nn.relu(z + bi_ref[...])
    z2 = jax.lax.dot(z, wm_ref[...], preferred_element_type=jnp.float32)
    z2 = jax.nn.relu(z2 + bm_ref[...]) + z
    o = jax.lax.dot(z2, wo_ref[...], preferred_element_type=jnp.float32)
    o_ref[...] = o + bo_ref[...]


def _decoder(x, p, pre):
    n = x.shape[0]
    ti = min(512, n)
    return pl.pallas_call(
        _dec_body,
        grid=(n // ti,),
        in_specs=[
            pl.BlockSpec((ti, D), lambda i: (i, 0)),
            pl.BlockSpec((D, H), lambda i: (0, 0)),
            pl.BlockSpec((1, H), lambda i: (0, 0)),
            pl.BlockSpec((H, H), lambda i: (0, 0)),
            pl.BlockSpec((1, H), lambda i: (0, 0)),
            pl.BlockSpec((H, D), lambda i: (0, 0)),
            pl.BlockSpec((1, D), lambda i: (0, 0)),
        ],
        out_specs=pl.BlockSpec((ti, D), lambda i: (i, 0)),
        out_shape=jax.ShapeDtypeStruct((n, D), jnp.float32),
    )(x, p[pre + 'W_in'], p[pre + 'b_in'].reshape(1, H),
      p[pre + 'W_mid'], p[pre + 'b_mid'].reshape(1, H),
      p[pre + 'W_out'], p[pre + 'b_out'].reshape(1, D))


def kernel(x_0, x_1, x_2, a0, a1, coa2, b1, b2, params):
    p = params

    hs0, qs0, _ = _proj(x_0, p['Ws_b1_1'], p['as_b1_1'], p['as_b1_1'])
    ht1, _, kt1 = _proj(x_1, p['Wt_b1_1'], p['at_b1_1'], p['at_b1_1'])
    s0, t1 = _dual_flash(qs0, kt1.T, b1, hs0, ht1)
    hs1, qs1, _ = _proj(x_1, p['Ws_b2_1'], p['as_b2_1'], p['as_b2_1'])
    ht2, _, kt2 = _proj(x_2, p['Wt_b2_1'], p['at_b2_1'], p['at_b2_1'])
    x1_l1, x2_l1 = _dual_flash(qs1, kt2.T, b2, hs1, ht2,
                               extra_s=t1, leaky_s=True, leaky_t=True)
    h0a, q0a, k0a = _proj(x_0, p['W_a0_1'], p['aq_a0_1'], p['ak_a0_1'])
    x0_l1 = _row_flash(q0a, k0a.T, a0, h0a, extra=s0, leaky_out=True)

    hsb1, qsb1, _ = _proj(x0_l1, p['Ws_b1_2'], p['as_b1_2'], p['as_b1_2'])
    ktb1 = _projvec(x1_l1, p['Wt_b1_2'], p['at_b1_2'])
    t1_2 = _col_flash(qsb1, ktb1.T, b1, hsb1)
    hsb2, qsb2, _ = _proj(x1_l1, p['Ws_b2_2'], p['as_b2_2'], p['as_b2_2'])
    ktb2 = _projvec(x2_l1, p['Wt_b2_2'], p['at_b2_2'])
    t2_2 = _col_flash(qsb2, ktb2.T, b2, hsb2)

    h0p, q0p, k0p = _proj(x0_l1, p['W_a0_2'], p['aq_a0_2'], p['ak_a0_2'])
    h0 = _row_flash(q0p, k0p.T, a0, h0p, residual=x_0, leaky_out=True)
    h1p, q1p, k1p = _proj(x1_l1, p['W_a1_2'], p['aq_a1_2'], p['ak_a1_2'])
    h1 = _row_flash(q1p, k1p.T, a1, h1p, extra=t1_2, residual=x_1,
                    leaky_out=True)
    h2p, q2p, k2p = _proj(x2_l1, p['W_coa2_2'], p['aq_coa2_2'], p['ak_coa2_2'])
    h2 = _row_flash(q2p, k2p.T, coa2, h2p, extra=t2_2, residual=x_2,
                    leaky_out=True)

    return (_decoder(h0, p, 'd0_'),
            _decoder(h1, p, 'd1_'),
            _decoder(h2, p, 'd2_'))


# speedup vs baseline: 1.1064x; 1.1064x over previous
import functools

import jax
import jax.numpy as jnp
from jax.experimental import pallas as pl
from jax.experimental.pallas import tpu as pltpu

D = 128
H = 256
THRESH = 0.99
SLOPE = 0.2


def _lk(x):
    return jnp.where(x >= 0, x, SLOPE * x)


def _proj_body(x_ref, w_ref, aq_ref, ak_ref,
               h_ref, q_ref, k_ref, qm_ref, km_ref, hs_ref):
    i = pl.program_id(0)
    h = jax.lax.dot(x_ref[...], w_ref[...], preferred_element_type=jnp.float32)
    h_ref[...] = h
    q = jax.lax.dot(h, aq_ref[...], preferred_element_type=jnp.float32)
    k = jax.lax.dot(h, ak_ref[...], preferred_element_type=jnp.float32)
    q_ref[...] = q
    k_ref[...] = k

    @pl.when(i == 0)
    def _():
        qm_ref[...] = jnp.full((1, 1), -jnp.inf, jnp.float32)
        km_ref[...] = jnp.full((1, 1), -jnp.inf, jnp.float32)
        hs_ref[...] = jnp.zeros(hs_ref.shape, jnp.float32)

    qm_ref[...] = jnp.maximum(qm_ref[...], jnp.max(q, keepdims=True))
    km_ref[...] = jnp.maximum(km_ref[...], jnp.max(k, keepdims=True))
    hs_ref[...] = hs_ref[...] + jnp.sum(h, axis=0, keepdims=True)


def _proj(x, w, aq, ak):
    n = x.shape[0]
    ti = min(512, n)
    return pl.pallas_call(
        _proj_body,
        grid=(n // ti,),
        in_specs=[
            pl.BlockSpec((ti, D), lambda i: (i, 0)),
            pl.BlockSpec((D, D), lambda i: (0, 0)),
            pl.BlockSpec((D, 1), lambda i: (0, 0)),
            pl.BlockSpec((D, 1), lambda i: (0, 0)),
        ],
        out_specs=[
            pl.BlockSpec((ti, D), lambda i: (i, 0)),
            pl.BlockSpec((ti, 1), lambda i: (i, 0)),
            pl.BlockSpec((ti, 1), lambda i: (i, 0)),
            pl.BlockSpec((1, 1), lambda i: (0, 0)),
            pl.BlockSpec((1, 1), lambda i: (0, 0)),
            pl.BlockSpec((1, D), lambda i: (0, 0)),
        ],
        out_shape=[
            jax.ShapeDtypeStruct((n, D), jnp.float32),
            jax.ShapeDtypeStruct((n, 1), jnp.float32),
            jax.ShapeDtypeStruct((n, 1), jnp.float32),
            jax.ShapeDtypeStruct((1, 1), jnp.float32),
            jax.ShapeDtypeStruct((1, 1), jnp.float32),
            jax.ShapeDtypeStruct((1, D), jnp.float32),
        ],
    )(x, w, aq.reshape(D, 1), ak.reshape(D, 1))


def _projvec_body(x_ref, w_ref, av_ref, k_ref):
    wv = jax.lax.dot(w_ref[...], av_ref[...], preferred_element_type=jnp.float32)
    k_ref[...] = jax.lax.dot(x_ref[...], wv, preferred_element_type=jnp.float32)


def _projvec(x, w, av):
    n = x.shape[0]
    ti = min(512, n)
    return pl.pallas_call(
        _projvec_body,
        grid=(n // ti,),
        in_specs=[
            pl.BlockSpec((ti, D), lambda i: (i, 0)),
            pl.BlockSpec((D, D), lambda i: (0, 0)),
            pl.BlockSpec((D, 1), lambda i: (0, 0)),
        ],
        out_specs=pl.BlockSpec((ti, 1), lambda i: (i, 0)),
        out_shape=jax.ShapeDtypeStruct((n, 1), jnp.float32),
    )(x, w, av.reshape(D, 1))


def _row_flash_body(gj, tj, nc, has_extra, has_res, leaky_out, *refs):
    q_ref, k_ref, km_ref, a_ref, h_ref, hs_ref = refs[:6]
    idx = 6
    e_ref = r_ref = None
    if has_extra:
        e_ref = refs[idx]
        idx += 1
    if has_res:
        r_ref = refs[idx]
        idx += 1
    o_ref, l_ref = refs[idx], refs[idx + 1]
    j = pl.program_id(1)

    @pl.when(j == 0)
    def _():
        o_ref[...] = jnp.zeros(o_ref.shape, jnp.float32)
        l_ref[...] = jnp.zeros(l_ref.shape, jnp.float32)

    q = q_ref[...]
    li = _lk(q + km_ref[...])
    z = q + k_ref[...]
    e = jnp.where(a_ref[...] > THRESH, jnp.exp(_lk(z) - li), 0.0)
    hb = h_ref[pl.ds(j * tj, tj), :]
    o_ref[...] += jax.lax.dot(e, hb, preferred_element_type=jnp.float32)
    l_ref[...] += jax.lax.dot(e, jnp.ones((tj, 1), jnp.float32),
                              preferred_element_type=jnp.float32)

    @pl.when(j == gj - 1)
    def _():
        l = l_ref[...]
        r = jnp.where(l > 0, o_ref[...] / (l + 1e-9),
                      hs_ref[...] / (nc + 1e-9))
        if has_extra:
            r = r + e_ref[...]
        if leaky_out:
            r = jnp.where(r >= 0, r, SLOPE * r)
        if has_res:
            r = r + r_ref[...]
        o_ref[...] = r


def _row_flash(q, kt, kmax, adj, h, hsum, extra=None, residual=None,
               leaky_out=False):
    nr, nc = adj.shape
    ti = min(256, nr)
    tj = min(512, nc)
    gi, gj = nr // ti, nc // tj
    inputs = [q, kt, kmax, adj, h, hsum]
    specs = [
        pl.BlockSpec((ti, 1), lambda i, j: (i, 0)),
        pl.BlockSpec((1, tj), lambda i, j: (0, j)),
        pl.BlockSpec((1, 1), lambda i, j: (0, 0)),
        pl.BlockSpec((ti, tj), lambda i, j: (i, j)),
        pl.BlockSpec((nc, D), lambda i, j: (0, 0)),
        pl.BlockSpec((1, D), lambda i, j: (0, 0)),
    ]
    if extra is not None:
        inputs.append(extra)
        specs.append(pl.BlockSpec((ti, D), lambda i, j: (i, 0)))
    if residual is not None:
        inputs.append(residual)
        specs.append(pl.BlockSpec((ti, D), lambda i, j: (i, 0)))
    body = functools.partial(_row_flash_body, gj, tj, float(nc),
                             extra is not None, residual is not None,
                             leaky_out)
    return pl.pallas_call(
        body,
        grid=(gi, gj),
        in_specs=specs,
        out_specs=pl.BlockSpec((ti, D), lambda i, j: (i, 0)),
        out_shape=jax.ShapeDtypeStruct((nr, D), jnp.float32),
        scratch_shapes=[pltpu.VMEM((ti, 1), jnp.float32)],
    )(*inputs)


def _col_flash_body(gs, ts, ns, q_ref, k_ref, qm_ref, a_ref, h_ref, hs_ref,
                    o_ref, l_ref):
    s_id = pl.program_id(1)

    @pl.when(s_id == 0)
    def _():
        o_ref[...] = jnp.zeros(o_ref.shape, jnp.float32)
        l_ref[...] = jnp.zeros(l_ref.shape, jnp.float32)

    k = k_ref[...]
    lt = _lk(qm_ref[...] + k)
    z = q_ref[...] + k
    e = jnp.where(a_ref[...] > THRESH, jnp.exp(_lk(z) - lt), 0.0)
    hb = h_ref[pl.ds(s_id * ts, ts), :]
    o_ref[...] += jax.lax.dot_general(
        e, hb, (((0,), (0,)), ((), ())), preferred_element_type=jnp.float32)
    l_ref[...] += jax.lax.dot(jnp.ones((1, ts), jnp.float32), e,
                              preferred_element_type=jnp.float32)

    @pl.when(s_id == gs - 1)
    def _():
        lc = l_ref[...].T
        o_ref[...] = jnp.where(lc > 0, o_ref[...] / (lc + 1e-9),
                               hs_ref[...] / (ns + 1e-9))


def _col_flash(q, kt, qmax, adj, hs, hssum):
    ns, nt = adj.shape
    ts = min(512, ns)
    tt = min(256, nt)
    gs, gt = ns // ts, nt // tt
    body = functools.partial(_col_flash_body, gs, ts, float(ns))
    return pl.pallas_call(
        body,
        grid=(gt, gs),
        in_specs=[
            pl.BlockSpec((ts, 1), lambda t, s: (s, 0)),
            pl.BlockSpec((1, tt), lambda t, s: (0, t)),
            pl.BlockSpec((1, 1), lambda t, s: (0, 0)),
            pl.BlockSpec((ts, tt), lambda t, s: (s, t)),
            pl.BlockSpec((ns, D), lambda t, s: (0, 0)),
            pl.BlockSpec((1, D), lambda t, s: (0, 0)),
        ],
        out_specs=pl.BlockSpec((tt, D), lambda t, s: (t, 0)),
        out_shape=jax.ShapeDtypeStruct((nt, D), jnp.float32),
        scratch_shapes=[pltpu.VMEM((1, tt), jnp.float32)],
    )(q, kt, qmax, adj, hs, hssum)


def _dual_body(gi, gj, tj, ns, nt, has_extra_s, leaky_s, leaky_t, *refs):
    (q_ref, k_ref, km_ref, qm_ref, a_ref, hs_ref, ht_ref,
     hss_ref, hts_ref) = refs[:9]
    idx = 9
    es_ref = None
    if has_extra_s:
        es_ref = refs[idx]
        idx += 1
    os_ref, ot_ref = refs[idx], refs[idx + 1]
    lr_ref, lc_ref = refs[idx + 2], refs[idx + 3]
    i = pl.program_id(0)
    j = pl.program_id(1)
    cds = pl.ds(j * tj, tj)

    @pl.when(j == 0)
    def _():
        os_ref[...] = jnp.zeros(os_ref.shape, jnp.float32)
        lr_ref[...] = jnp.zeros(lr_ref.shape, jnp.float32)

    @pl.when(i == 0)
    def _():
        ot_ref[cds, :] = jnp.zeros((tj, D), jnp.float32)
        lc_ref[:, cds] = jnp.zeros((1, tj), jnp.float32)

    q = q_ref[...]
    k = k_ref[...]
    li = _lk(q + km_ref[...])
    lt = _lk(qm_ref[...] + k)
    lz = _lk(q + k)
    mask = a_ref[...] > THRESH
    e_r = jnp.where(mask, jnp.exp(lz - li), 0.0)
    e_c = jnp.where(mask, jnp.exp(lz - lt), 0.0)

    os_ref[...] += jax.lax.dot(e_r, ht_ref[cds, :],
                               preferred_element_type=jnp.float32)
    lr_ref[...] += jax.lax.dot(e_r, jnp.ones((tj, 1), jnp.float32),
                               preferred_element_type=jnp.float32)

    @pl.when(j == gj - 1)
    def _():
        l = lr_ref[...]
        r = jnp.where(l > 0, os_ref[...] / (l + 1e-9),
                      hts_ref[...] / (nt + 1e-9))
        if has_extra_s:
            r = r + es_ref[...]
        if leaky_s:
            r = jnp.where(r >= 0, r, SLOPE * r)
        os_ref[...] = r

    ot_ref[cds, :] += jax.lax.dot_general(
        e_c, hs_ref[...], (((0,), (0,)), ((), ())),
        preferred_element_type=jnp.float32)
    lc_ref[:, cds] += jax.lax.dot(jnp.ones((1, e_c.shape[0]), jnp.float32),
                                  e_c, preferred_element_type=jnp.float32)

    @pl.when(i == gi - 1)
    def _():
        lc = lc_ref[:, cds].T
        r = jnp.where(lc > 0, ot_ref[cds, :] / (lc + 1e-9),
                      hss_ref[...] / (ns + 1e-9))
        if leaky_t:
            r = jnp.where(r >= 0, r, SLOPE * r)
        ot_ref[cds, :] = r


def _dual_flash(q, kt, kmax, qmax, adj, hs, ht, hssum, htsum,
                extra_s=None, leaky_s=False, leaky_t=False):
    ns, nt = adj.shape
    ti = min(256, ns)
    tj = min(512, nt)
    gi, gj = ns // ti, nt // tj
    inputs = [q, kt, kmax, qmax, adj, hs, ht, hssum, htsum]
    specs = [
        pl.BlockSpec((ti, 1), lambda i, j: (i, 0)),
        pl.BlockSpec((1, tj), lambda i, j: (0, j)),
        pl.BlockSpec((1, 1), lambda i, j: (0, 0)),
        pl.BlockSpec((1, 1), lambda i, j: (0, 0)),
        pl.BlockSpec((ti, tj), lambda i, j: (i, j)),
        pl.BlockSpec((ti, D), lambda i, j: (i, 0)),
        pl.BlockSpec((nt, D), lambda i, j: (0, 0)),
        pl.BlockSpec((1, D), lambda i, j: (0, 0)),
        pl.BlockSpec((1, D), lambda i, j: (0, 0)),
    ]
    if extra_s is not None:
        inputs.append(extra_s)
        specs.append(pl.BlockSpec((ti, D), lambda i, j: (i, 0)))
    body = functools.partial(_dual_body, gi, gj, tj, float(ns), float(nt),
                             extra_s is not None, leaky_s, leaky_t)
    return pl.pallas_call(
        body,
        grid=(gi, gj),
        in_specs=specs,
        out_specs=[
            pl.BlockSpec((ti, D), lambda i, j: (i, 0)),
            pl.BlockSpec((nt, D), lambda i, j: (0, 0)),
        ],
        out_shape=[
            jax.ShapeDtypeStruct((ns, D), jnp.float32),
            jax.ShapeDtypeStruct((nt, D), jnp.float32),
        ],
        scratch_shapes=[
            pltpu.VMEM((ti, 1), jnp.float32),
            pltpu.VMEM((1, nt), jnp.float32),
        ],
    )(*inputs)


def _dec_body(x_ref, wi_ref, bi_ref, wm_ref, bm_ref, wo_ref, bo_ref, o_ref):
    z = jax.lax.dot(x_ref[...], wi_ref[...], preferred_element_type=jnp.float32)
    z = jax.nn.relu(z + bi_ref[...])
    z2 = jax.lax.dot(z, wm_ref[...], preferred_element_type=jnp.float32)
    z2 = jax.nn.relu(z2 + bm_ref[...]) + z
    o = jax.lax.dot(z2, wo_ref[...], preferred_element_type=jnp.float32)
    o_ref[...] = o + bo_ref[...]


def _decoder(x, p, pre):
    n = x.shape[0]
    ti = min(512, n)
    return pl.pallas_call(
        _dec_body,
        grid=(n // ti,),
        in_specs=[
            pl.BlockSpec((ti, D), lambda i: (i, 0)),
            pl.BlockSpec((D, H), lambda i: (0, 0)),
            pl.BlockSpec((1, H), lambda i: (0, 0)),
            pl.BlockSpec((H, H), lambda i: (0, 0)),
            pl.BlockSpec((1, H), lambda i: (0, 0)),
            pl.BlockSpec((H, D), lambda i: (0, 0)),
            pl.BlockSpec((1, D), lambda i: (0, 0)),
        ],
        out_specs=pl.BlockSpec((ti, D), lambda i: (i, 0)),
        out_shape=jax.ShapeDtypeStruct((n, D), jnp.float32),
    )(x, p[pre + 'W_in'], p[pre + 'b_in'].reshape(1, H),
      p[pre + 'W_mid'], p[pre + 'b_mid'].reshape(1, H),
      p[pre + 'W_out'], p[pre + 'b_out'].reshape(1, D))


def kernel(x_0, x_1, x_2, a0, a1, coa2, b1, b2, params):
    p = params

    hs0, qs0, _, qm0, _, hsum0 = _proj(x_0, p['Ws_b1_1'],
                                       p['as_b1_1'], p['as_b1_1'])
    ht1, _, kt1, _, km1, hsum1 = _proj(x_1, p['Wt_b1_1'],
                                       p['at_b1_1'], p['at_b1_1'])
    s0, t1 = _dual_flash(qs0, kt1.T, km1, qm0, b1, hs0, ht1, hsum0, hsum1)
    hs1, qs1, _, qm1, _, hsum1b = _proj(x_1, p['Ws_b2_1'],
                                        p['as_b2_1'], p['as_b2_1'])
    ht2, _, kt2, _, km2, hsum2 = _proj(x_2, p['Wt_b2_1'],
                                       p['at_b2_1'], p['at_b2_1'])
    x1_l1, x2_l1 = _dual_flash(qs1, kt2.T, km2, qm1, b2, hs1, ht2,
                               hsum1b, hsum2,
                               extra_s=t1, leaky_s=True, leaky_t=True)
    h0a, q0a, k0a, _, km0a, hsum0a = _proj(x_0, p['W_a0_1'],
                                           p['aq_a0_1'], p['ak_a0_1'])
    x0_l1 = _row_flash(q0a, k0a.T, km0a, a0, h0a, hsum0a,
                       extra=s0, leaky_out=True)

    hsb1, qsb1, _, qmb1, _, hsumb1 = _proj(x0_l1, p['Ws_b1_2'],
                                           p['as_b1_2'], p['as_b1_2'])
    ktb1 = _projvec(x1_l1, p['Wt_b1_2'], p['at_b1_2'])
    t1_2 = _col_flash(qsb1, ktb1.T, qmb1, b1, hsb1, hsumb1)
    hsb2, qsb2, _, qmb2, _, hsumb2 = _proj(x1_l1, p['Ws_b2_2'],
                                           p['as_b2_2'], p['as_b2_2'])
    ktb2 = _projvec(x2_l1, p['Wt_b2_2'], p['at_b2_2'])
    t2_2 = _col_flash(qsb2, ktb2.T, qmb2, b2, hsb2, hsumb2)

    h0p, q0p, k0p, _, km0p, hsum0p = _proj(x0_l1, p['W_a0_2'],
                                           p['aq_a0_2'], p['ak_a0_2'])
    h0 = _row_flash(q0p, k0p.T, km0p, a0, h0p, hsum0p,
                    residual=x_0, leaky_out=True)
    h1p, q1p, k1p, _, km1p, hsum1p = _proj(x1_l1, p['W_a1_2'],
                                           p['aq_a1_2'], p['ak_a1_2'])
    h1 = _row_flash(q1p, k1p.T, km1p, a1, h1p, hsum1p,
                    extra=t1_2, residual=x_1, leaky_out=True)
    h2p, q2p, k2p, _, km2p, hsum2p = _proj(x2_l1, p['W_coa2_2'],
                                           p['aq_coa2_2'], p['ak_coa2_2'])
    h2 = _row_flash(q2p, k2p.T, km2p, coa2, h2p, hsum2p,
                    extra=t2_2, residual=x_2, leaky_out=True)

    return (_decoder(h0, p, 'd0_'),
            _decoder(h1, p, 'd1_'),
            _decoder(h2, p, 'd2_'))


# speedup vs baseline: 1.7576x; 1.5886x over previous
import functools

import jax
import jax.numpy as jnp
from jax.experimental import pallas as pl
from jax.experimental.pallas import tpu as pltpu

D = 128
H = 256
THRESH = 0.99
SLOPE = 0.2


def _lk(x):
    return jnp.maximum(x, SLOPE * x)


def _proj_body(x_ref, w_ref, aq_ref, ak_ref,
               h_ref, q_ref, k_ref, qm_ref, km_ref, hs_ref):
    i = pl.program_id(0)
    h = jax.lax.dot(x_ref[...], w_ref[...], preferred_element_type=jnp.float32)
    h_ref[...] = h
    q = jax.lax.dot(h, aq_ref[...], preferred_element_type=jnp.float32)
    k = jax.lax.dot(h, ak_ref[...], preferred_element_type=jnp.float32)
    q_ref[...] = q
    k_ref[...] = k

    @pl.when(i == 0)
    def _():
        qm_ref[...] = jnp.full((1, 1), -jnp.inf, jnp.float32)
        km_ref[...] = jnp.full((1, 1), -jnp.inf, jnp.float32)
        hs_ref[...] = jnp.zeros(hs_ref.shape, jnp.float32)

    qm_ref[...] = jnp.maximum(qm_ref[...], jnp.max(q, keepdims=True))
    km_ref[...] = jnp.maximum(km_ref[...], jnp.max(k, keepdims=True))
    hs_ref[...] = hs_ref[...] + jnp.sum(h, axis=0, keepdims=True)


def _proj(x, w, aq, ak):
    n = x.shape[0]
    ti = min(512, n)
    return pl.pallas_call(
        _proj_body,
        grid=(n // ti,),
        in_specs=[
            pl.BlockSpec((ti, D), lambda i: (i, 0)),
            pl.BlockSpec((D, D), lambda i: (0, 0)),
            pl.BlockSpec((D, 1), lambda i: (0, 0)),
            pl.BlockSpec((D, 1), lambda i: (0, 0)),
        ],
        out_specs=[
            pl.BlockSpec((ti, D), lambda i: (i, 0)),
            pl.BlockSpec((ti, 1), lambda i: (i, 0)),
            pl.BlockSpec((ti, 1), lambda i: (i, 0)),
            pl.BlockSpec((1, 1), lambda i: (0, 0)),
            pl.BlockSpec((1, 1), lambda i: (0, 0)),
            pl.BlockSpec((1, D), lambda i: (0, 0)),
        ],
        out_shape=[
            jax.ShapeDtypeStruct((n, D), jnp.float32),
            jax.ShapeDtypeStruct((n, 1), jnp.float32),
            jax.ShapeDtypeStruct((n, 1), jnp.float32),
            jax.ShapeDtypeStruct((1, 1), jnp.float32),
            jax.ShapeDtypeStruct((1, 1), jnp.float32),
            jax.ShapeDtypeStruct((1, D), jnp.float32),
        ],
    )(x, w, aq.reshape(D, 1), ak.reshape(D, 1))


def _projvec_body(x_ref, w_ref, av_ref, k_ref):
    wv = jax.lax.dot(w_ref[...], av_ref[...], preferred_element_type=jnp.float32)
    k_ref[...] = jax.lax.dot(x_ref[...], wv, preferred_element_type=jnp.float32)


def _projvec(x, w, av):
    n = x.shape[0]
    ti = min(512, n)
    return pl.pallas_call(
        _projvec_body,
        grid=(n // ti,),
        in_specs=[
            pl.BlockSpec((ti, D), lambda i: (i, 0)),
            pl.BlockSpec((D, D), lambda i: (0, 0)),
            pl.BlockSpec((D, 1), lambda i: (0, 0)),
        ],
        out_specs=pl.BlockSpec((ti, 1), lambda i: (i, 0)),
        out_shape=jax.ShapeDtypeStruct((n, 1), jnp.float32),
    )(x, w, av.reshape(D, 1))


def _row_flash_body(gj, tj, nc, has_extra, has_res, leaky_out, *refs):
    q_ref, k_ref, km_ref, a_ref, h_ref, hs_ref = refs[:6]
    idx = 6
    e_ref = r_ref = None
    if has_extra:
        e_ref = refs[idx]
        idx += 1
    if has_res:
        r_ref = refs[idx]
        idx += 1
    o_ref, l_ref = refs[idx], refs[idx + 1]
    j = pl.program_id(1)

    @pl.when(j == 0)
    def _():
        o_ref[...] = jnp.zeros(o_ref.shape, jnp.float32)
        l_ref[...] = jnp.zeros(l_ref.shape, jnp.float32)

    q = q_ref[...]
    li = _lk(q + km_ref[...])
    z = q + k_ref[...]
    e = jnp.where(a_ref[...] > THRESH, jnp.exp(_lk(z) - li), 0.0)
    hb = h_ref[pl.ds(j * tj, tj), :]
    o_ref[...] += jax.lax.dot(e, hb, preferred_element_type=jnp.float32)
    l_ref[...] += jax.lax.dot(e, jnp.ones((tj, 1), jnp.float32),
                              preferred_element_type=jnp.float32)

    @pl.when(j == gj - 1)
    def _():
        l = l_ref[...]
        r = jnp.where(l > 0, o_ref[...] / (l + 1e-9),
                      hs_ref[...] / (nc + 1e-9))
        if has_extra:
            r = r + e_ref[...]
        if leaky_out:
            r = jnp.maximum(r, SLOPE * r)
        if has_res:
            r = r + r_ref[...]
        o_ref[...] = r


def _row_flash(q, kt, kmax, adj, h, hsum, extra=None, residual=None,
               leaky_out=False):
    nr, nc = adj.shape
    ti = min(512, nr)
    tj = min(1024, nc)
    gi, gj = nr // ti, nc // tj
    inputs = [q, kt, kmax, adj, h, hsum]
    specs = [
        pl.BlockSpec((ti, 1), lambda i, j: (i, 0)),
        pl.BlockSpec((1, tj), lambda i, j: (0, j)),
        pl.BlockSpec((1, 1), lambda i, j: (0, 0)),
        pl.BlockSpec((ti, tj), lambda i, j: (i, j)),
        pl.BlockSpec((nc, D), lambda i, j: (0, 0)),
        pl.BlockSpec((1, D), lambda i, j: (0, 0)),
    ]
    if extra is not None:
        inputs.append(extra)
        specs.append(pl.BlockSpec((ti, D), lambda i, j: (i, 0)))
    if residual is not None:
        inputs.append(residual)
        specs.append(pl.BlockSpec((ti, D), lambda i, j: (i, 0)))
    body = functools.partial(_row_flash_body, gj, tj, float(nc),
                             extra is not None, residual is not None,
                             leaky_out)
    return pl.pallas_call(
        body,
        grid=(gi, gj),
        in_specs=specs,
        out_specs=pl.BlockSpec((ti, D), lambda i, j: (i, 0)),
        out_shape=jax.ShapeDtypeStruct((nr, D), jnp.float32),
        scratch_shapes=[pltpu.VMEM((ti, 1), jnp.float32)],
    )(*inputs)


def _col_flash_body(gs, ts, ns, q_ref, k_ref, qm_ref, a_ref, h_ref, hs_ref,
                    o_ref, acc_ref, l_ref, hsT_ref):
    t = pl.program_id(0)
    s_id = pl.program_id(1)
    sds = pl.ds(s_id * ts, ts)

    @pl.when(t == 0)
    def _():
        hsT_ref[:, sds] = h_ref[sds, :].T

    @pl.when(s_id == 0)
    def _():
        acc_ref[...] = jnp.zeros(acc_ref.shape, jnp.float32)
        l_ref[...] = jnp.zeros(l_ref.shape, jnp.float32)

    k = k_ref[...]
    lt = _lk(qm_ref[...] + k)
    z = q_ref[...] + k
    e = jnp.where(a_ref[...] > THRESH, jnp.exp(_lk(z) - lt), 0.0)
    acc_ref[...] += jax.lax.dot(hsT_ref[:, sds], e,
                                preferred_element_type=jnp.float32)
    l_ref[...] += jax.lax.dot(jnp.ones((1, ts), jnp.float32), e,
                              preferred_element_type=jnp.float32)

    @pl.when(s_id == gs - 1)
    def _():
        l = l_ref[...]
        r = jnp.where(l > 0, acc_ref[...] / (l + 1e-9),
                      hs_ref[...].T / (ns + 1e-9))
        o_ref[...] = r.T


def _col_flash(q, kt, qmax, adj, hs, hssum):
    ns, nt = adj.shape
    ts = min(1024, ns)
    tt = min(512, nt)
    gs, gt = ns // ts, nt // tt
    body = functools.partial(_col_flash_body, gs, ts, float(ns))
    return pl.pallas_call(
        body,
        grid=(gt, gs),
        in_specs=[
            pl.BlockSpec((ts, 1), lambda t, s: (s, 0)),
            pl.BlockSpec((1, tt), lambda t, s: (0, t)),
            pl.BlockSpec((1, 1), lambda t, s: (0, 0)),
            pl.BlockSpec((ts, tt), lambda t, s: (s, t)),
            pl.BlockSpec((ns, D), lambda t, s: (0, 0)),
            pl.BlockSpec((1, D), lambda t, s: (0, 0)),
        ],
        out_specs=pl.BlockSpec((tt, D), lambda t, s: (t, 0)),
        out_shape=jax.ShapeDtypeStruct((nt, D), jnp.float32),
        scratch_shapes=[
            pltpu.VMEM((D, tt), jnp.float32),
            pltpu.VMEM((1, tt), jnp.float32),
            pltpu.VMEM((D, ns), jnp.float32),
        ],
    )(q, kt, qmax, adj, hs, hssum)


def _dual_body(gi, gj, tj, ns, nt, has_extra_s, leaky_s, leaky_t, *refs):
    (q_ref, qt_ref, k_ref, km_ref, qm_ref, a_ref, hs_ref, ht_ref,
     hss_ref, hts_ref) = refs[:10]
    idx = 10
    es_ref = None
    if has_extra_s:
        es_ref = refs[idx]
        idx += 1
    os_ref, ot_ref = refs[idx], refs[idx + 1]
    lr_ref, lc_ref, otT_ref, hsT_ref = refs[idx + 2:idx + 6]
    i = pl.program_id(0)
    j = pl.program_id(1)
    cds = pl.ds(j * tj, tj)

    km = km_ref[...]
    qm = qm_ref[...]
    lmax = _lk(qm + km)
    q = q_ref[...]
    k = k_ref[...]
    li = _lk(q + km)
    lt = _lk(qm + k)

    @pl.when(j == 0)
    def _():
        os_ref[...] = jnp.zeros(os_ref.shape, jnp.float32)
        lr_ref[...] = jnp.zeros(lr_ref.shape, jnp.float32)
        wt = jnp.exp(_lk(qt_ref[...] + km) - lmax)
        hsT_ref[...] = hs_ref[...].T * wt

    @pl.when(i == 0)
    def _():
        otT_ref[:, cds] = jnp.zeros((D, tj), jnp.float32)
        lc_ref[:, cds] = jnp.zeros((1, tj), jnp.float32)

    z = q + k
    e = jnp.where(a_ref[...] > THRESH, jnp.exp(_lk(z) - li), 0.0)

    os_ref[...] += jax.lax.dot(e, ht_ref[cds, :],
                               preferred_element_type=jnp.float32)
    lr_ref[...] += jax.lax.dot(e, jnp.ones((tj, 1), jnp.float32),
                               preferred_element_type=jnp.float32)

    @pl.when(j == gj - 1)
    def _():
        l = lr_ref[...]
        r = jnp.where(l > 0, os_ref[...] / (l + 1e-9),
                      hts_ref[...] / (nt + 1e-9))
        if has_extra_s:
            r = r + es_ref[...]
        if leaky_s:
            r = jnp.maximum(r, SLOPE * r)
        os_ref[...] = r

    wt = jnp.exp(_lk(qt_ref[...] + km) - lmax)
    otT_ref[:, cds] += jax.lax.dot(hsT_ref[...], e,
                                   preferred_element_type=jnp.float32)
    lc_ref[:, cds] += jax.lax.dot(wt, e, preferred_element_type=jnp.float32)

    @pl.when(i == gi - 1)
    def _():
        f = jnp.exp(lmax - lt)
        lf = lc_ref[:, cds] * f
        rt = jnp.where(lf > 0, (otT_ref[:, cds] * f) / (lf + 1e-9),
                       hss_ref[...].T / (ns + 1e-9))
        if leaky_t:
            rt = jnp.maximum(rt, SLOPE * rt)
        ot_ref[cds, :] = rt.T


def _dual_flash(q, qt, kt, kmax, qmax, adj, hs, ht, hssum, htsum,
                extra_s=None, leaky_s=False, leaky_t=False):
    ns, nt = adj.shape
    ti = min(512, ns)
    tj = min(1024, nt)
    gi, gj = ns // ti, nt // tj
    inputs = [q, qt, kt, kmax, qmax, adj, hs, ht, hssum, htsum]
    specs = [
        pl.BlockSpec((ti, 1), lambda i, j: (i, 0)),
        pl.BlockSpec((1, ti), lambda i, j: (0, i)),
        pl.BlockSpec((1, tj), lambda i, j: (0, j)),
        pl.BlockSpec((1, 1), lambda i, j: (0, 0)),
        pl.BlockSpec((1, 1), lambda i, j: (0, 0)),
        pl.BlockSpec((ti, tj), lambda i, j: (i, j)),
        pl.BlockSpec((ti, D), lambda i, j: (i, 0)),
        pl.BlockSpec((nt, D), lambda i, j: (0, 0)),
        pl.BlockSpec((1, D), lambda i, j: (0, 0)),
        pl.BlockSpec((1, D), lambda i, j: (0, 0)),
    ]
    if extra_s is not None:
        inputs.append(extra_s)
        specs.append(pl.BlockSpec((ti, D), lambda i, j: (i, 0)))
    body = functools.partial(_dual_body, gi, gj, tj, float(ns), float(nt),
                             extra_s is not None, leaky_s, leaky_t)
    return pl.pallas_call(
        body,
        grid=(gi, gj),
        in_specs=specs,
        out_specs=[
            pl.BlockSpec((ti, D), lambda i, j: (i, 0)),
            pl.BlockSpec((nt, D), lambda i, j: (0, 0)),
        ],
        out_shape=[
            jax.ShapeDtypeStruct((ns, D), jnp.float32),
            jax.ShapeDtypeStruct((nt, D), jnp.float32),
        ],
        scratch_shapes=[
            pltpu.VMEM((ti, 1), jnp.float32),
            pltpu.VMEM((1, nt), jnp.float32),
            pltpu.VMEM((D, nt), jnp.float32),
            pltpu.VMEM((D, ti), jnp.float32),
        ],
    )(*inputs)


def _dec_body(x_ref, wi_ref, bi_ref, wm_ref, bm_ref, wo_ref, bo_ref, o_ref):
    z = jax.lax.dot(x_ref[...], wi_ref[...], preferred_element_type=jnp.float32)
    z = jnp.maximum(z + bi_ref[...], 0.0)
    z2 = jax.lax.dot(z, wm_ref[...], preferred_element_type=jnp.float32)
    z2 = jnp.maximum(z2 + bm_ref[...], 0.0) + z
    o = jax.lax.dot(z2, wo_ref[...], preferred_element_type=jnp.float32)
    o_ref[...] = o + bo_ref[...]


def _decoder(x, p, pre):
    n = x.shape[0]
    ti = min(512, n)
    return pl.pallas_call(
        _dec_body,
        grid=(n // ti,),
        in_specs=[
            pl.BlockSpec((ti, D), lambda i: (i, 0)),
            pl.BlockSpec((D, H), lambda i: (0, 0)),
            pl.BlockSpec((1, H), lambda i: (0, 0)),
            pl.BlockSpec((H, H), lambda i: (0, 0)),
            pl.BlockSpec((1, H), lambda i: (0, 0)),
            pl.BlockSpec((H, D), lambda i: (0, 0)),
            pl.BlockSpec((1, D), lambda i: (0, 0)),
        ],
        out_specs=pl.BlockSpec((ti, D), lambda i: (i, 0)),
        out_shape=jax.ShapeDtypeStruct((n, D), jnp.float32),
    )(x, p[pre + 'W_in'], p[pre + 'b_in'].reshape(1, H),
      p[pre + 'W_mid'], p[pre + 'b_mid'].reshape(1, H),
      p[pre + 'W_out'], p[pre + 'b_out'].reshape(1, D))


def kernel(x_0, x_1, x_2, a0, a1, coa2, b1, b2, params):
    p = params

    hs0, qs0, _, qm0, _, hsum0 = _proj(x_0, p['Ws_b1_1'],
                                       p['as_b1_1'], p['as_b1_1'])
    ht1, _, kt1, _, km1, hsum1 = _proj(x_1, p['Wt_b1_1'],
                                       p['at_b1_1'], p['at_b1_1'])
    s0, t1 = _dual_flash(qs0, qs0.T, kt1.T, km1, qm0, b1, hs0, ht1,
                         hsum0, hsum1)
    hs1, qs1, _, qm1, _, hsum1b = _proj(x_1, p['Ws_b2_1'],
                                        p['as_b2_1'], p['as_b2_1'])
    ht2, _, kt2, _, km2, hsum2 = _proj(x_2, p['Wt_b2_1'],
                                       p['at_b2_1'], p['at_b2_1'])
    x1_l1, x2_l1 = _dual_flash(qs1, qs1.T, kt2.T, km2, qm1, b2, hs1, ht2,
                               hsum1b, hsum2,
                               extra_s=t1, leaky_s=True, leaky_t=True)
    h0a, q0a, k0a, _, km0a, hsum0a = _proj(x_0, p['W_a0_1'],
                                           p['aq_a0_1'], p['ak_a0_1'])
    x0_l1 = _row_flash(q0a, k0a.T, km0a, a0, h0a, hsum0a,
                       extra=s0, leaky_out=True)

    hsb1, qsb1, _, qmb1, _, hsumb1 = _proj(x0_l1, p['Ws_b1_2'],
                                           p['as_b1_2'], p['as_b1_2'])
    ktb1 = _projvec(x1_l1, p['Wt_b1_2'], p['at_b1_2'])
    t1_2 = _col_flash(qsb1, ktb1.T, qmb1, b1, hsb1, hsumb1)
    hsb2, qsb2, _, qmb2, _, hsumb2 = _proj(x1_l1, p['Ws_b2_2'],
                                           p['as_b2_2'], p['as_b2_2'])
    ktb2 = _projvec(x2_l1, p['Wt_b2_2'], p['at_b2_2'])
    t2_2 = _col_flash(qsb2, ktb2.T, qmb2, b2, hsb2, hsumb2)

    h0p, q0p, k0p, _, km0p, hsum0p = _proj(x0_l1, p['W_a0_2'],
                                           p['aq_a0_2'], p['ak_a0_2'])
    h0 = _row_flash(q0p, k0p.T, km0p, a0, h0p, hsum0p,
                    residual=x_0, leaky_out=True)
    h1p, q1p, k1p, _, km1p, hsum1p = _proj(x1_l1, p['W_a1_2'],
                                           p['aq_a1_2'], p['ak_a1_2'])
    h1 = _row_flash(q1p, k1p.T, km1p, a1, h1p, hsum1p,
                    extra=t1_2, residual=x_1, leaky_out=True)
    h2p, q2p, k2p, _, km2p, hsum2p = _proj(x2_l1, p['W_coa2_2'],
                                           p['aq_coa2_2'], p['ak_coa2_2'])
    h2 = _row_flash(q2p, k2p.T, km2p, coa2, h2p, hsum2p,
                    extra=t2_2, residual=x_2, leaky_out=True)

    return (_decoder(h0, p, 'd0_'),
            _decoder(h1, p, 'd1_'),
            _decoder(h2, p, 'd2_'))


# speedup vs baseline: 2.5115x; 1.4290x over previous
import functools

import jax
import jax.numpy as jnp
from jax.experimental import pallas as pl
from jax.experimental.pallas import tpu as pltpu

D = 128
H = 256
THRESH = 0.99
SLOPE = 0.2


def _lk(x):
    return jnp.maximum(x, SLOPE * x)


def _dec_apply(r, wi, bi, wm, bm, wo, bo):
    z = jax.lax.dot(r, wi, preferred_element_type=jnp.float32)
    z = jnp.maximum(z + bi, 0.0)
    z2 = jax.lax.dot(z, wm, preferred_element_type=jnp.float32)
    z2 = jnp.maximum(z2 + bm, 0.0) + z
    o = jax.lax.dot(z2, wo, preferred_element_type=jnp.float32)
    return o + bo


def _row_body(gj, ti, tj, n, has_extra, has_res, leaky_out, has_dec, *refs):
    x_ref, w_ref, aq_ref, ak_ref, a_ref = refs[:5]
    idx = 5
    e_ref = r_ref = None
    if has_extra:
        e_ref = refs[idx]
        idx += 1
    if has_res:
        r_ref = refs[idx]
        idx += 1
    dec = None
    if has_dec:
        dec = refs[idx:idx + 6]
        idx += 6
    o_ref = refs[idx]
    idx += 1
    (h_ref, q_ref, kt_ref, km_ref, hsum_ref, l_ref, acc_ref) = refs[idx:]
    i = pl.program_id(0)
    j = pl.program_id(1)

    @pl.when((i == 0) & (j == 0))
    def _():
        h = jax.lax.dot(x_ref[...], w_ref[...],
                        preferred_element_type=jnp.float32)
        h_ref[...] = h
        q_ref[...] = jax.lax.dot(h, aq_ref[...],
                                 preferred_element_type=jnp.float32)
        k = jax.lax.dot(h, ak_ref[...], preferred_element_type=jnp.float32)
        kt_ref[...] = k.T
        km_ref[...] = jnp.max(k, keepdims=True)
        hsum_ref[...] = jnp.sum(h, axis=0, keepdims=True)

    @pl.when(j == 0)
    def _():
        acc_ref[...] = jnp.zeros(acc_ref.shape, jnp.float32)
        l_ref[...] = jnp.zeros(l_ref.shape, jnp.float32)

    rds = pl.ds(i * ti, ti)
    cds = pl.ds(j * tj, tj)
    q = q_ref[rds, :]
    li = _lk(q + km_ref[...])
    z = q + kt_ref[:, cds]
    e = jnp.where(a_ref[...] > THRESH, jnp.exp(_lk(z) - li), 0.0)
    acc_ref[...] += jax.lax.dot(e, h_ref[cds, :],
                                preferred_element_type=jnp.float32)
    l_ref[...] += jax.lax.dot(e, jnp.ones((tj, 1), jnp.float32),
                              preferred_element_type=jnp.float32)

    @pl.when(j == gj - 1)
    def _():
        l = l_ref[...]
        r = jnp.where(l > 0, acc_ref[...] / (l + 1e-9),
                      hsum_ref[...] / (n + 1e-9))
        if has_extra:
            r = r + e_ref[...]
        if leaky_out:
            r = jnp.maximum(r, SLOPE * r)
        if has_res:
            r = r + r_ref[...]
        if has_dec:
            r = _dec_apply(r, dec[0][...], dec[1][...], dec[2][...],
                           dec[3][...], dec[4][...], dec[5][...])
        o_ref[...] = r


def _row_flash(x, w, aq, ak, adj, extra=None, residual=None, leaky_out=False,
               dec=None):
    n = adj.shape[0]
    ti = min(512, n)
    tj = min(1024, n)
    gi, gj = n // ti, n // tj
    inputs = [x, w, aq.reshape(D, 1), ak.reshape(D, 1), adj]
    specs = [
        pl.BlockSpec((n, D), lambda i, j: (0, 0)),
        pl.BlockSpec((D, D), lambda i, j: (0, 0)),
        pl.BlockSpec((D, 1), lambda i, j: (0, 0)),
        pl.BlockSpec((D, 1), lambda i, j: (0, 0)),
        pl.BlockSpec((ti, tj), lambda i, j: (i, j)),
    ]
    if extra is not None:
        inputs.append(extra)
        specs.append(pl.BlockSpec((ti, D), lambda i, j: (i, 0)))
    if residual is not None:
        inputs.append(residual)
        specs.append(pl.BlockSpec((ti, D), lambda i, j: (i, 0)))
    if dec is not None:
        inputs.extend(dec)
        specs.extend([
            pl.BlockSpec((D, H), lambda i, j: (0, 0)),
            pl.BlockSpec((1, H), lambda i, j: (0, 0)),
            pl.BlockSpec((H, H), lambda i, j: (0, 0)),
            pl.BlockSpec((1, H), lambda i, j: (0, 0)),
            pl.BlockSpec((H, D), lambda i, j: (0, 0)),
            pl.BlockSpec((1, D), lambda i, j: (0, 0)),
        ])
    body = functools.partial(_row_body, gj, ti, tj, float(n),
                             extra is not None, residual is not None,
                             leaky_out, dec is not None)
    return pl.pallas_call(
        body,
        grid=(gi, gj),
        in_specs=specs,
        out_specs=pl.BlockSpec((ti, D), lambda i, j: (i, 0)),
        out_shape=jax.ShapeDtypeStruct((n, D), jnp.float32),
        scratch_shapes=[
            pltpu.VMEM((n, D), jnp.float32),
            pltpu.VMEM((n, 1), jnp.float32),
            pltpu.VMEM((1, n), jnp.float32),
            pltpu.VMEM((1, 1), jnp.float32),
            pltpu.VMEM((1, D), jnp.float32),
            pltpu.VMEM((ti, 1), jnp.float32),
            pltpu.VMEM((ti, D), jnp.float32),
        ],
    )(*inputs)


def _col_body(gs, ts, tt, ns,
              xs_ref, ws_ref, avs_ref, xt_ref, wt_ref, avt_ref, a_ref,
              o_ref, hsT_ref, q_ref, kt_ref, qm_ref, hsumT_ref,
              acc_ref, l_ref):
    t = pl.program_id(0)
    s_id = pl.program_id(1)

    @pl.when((t == 0) & (s_id == 0))
    def _():
        hs = jax.lax.dot(xs_ref[...], ws_ref[...],
                         preferred_element_type=jnp.float32)
        hsT_ref[...] = hs.T
        q = jax.lax.dot(hs, avs_ref[...], preferred_element_type=jnp.float32)
        q_ref[...] = q
        qm_ref[...] = jnp.max(q, keepdims=True)
        hsumT_ref[...] = jnp.sum(hs, axis=0, keepdims=True).T
        wv = jax.lax.dot(wt_ref[...], avt_ref[...],
                         preferred_element_type=jnp.float32)
        kt_ref[...] = jax.lax.dot(xt_ref[...], wv,
                                  preferred_element_type=jnp.float32).T

    @pl.when(s_id == 0)
    def _():
        acc_ref[...] = jnp.zeros(acc_ref.shape, jnp.float32)
        l_ref[...] = jnp.zeros(l_ref.shape, jnp.float32)

    sds = pl.ds(s_id * ts, ts)
    k = kt_ref[:, pl.ds(t * tt, tt)]
    lt = _lk(qm_ref[...] + k)
    z = q_ref[sds, :] + k
    e = jnp.where(a_ref[...] > THRESH, jnp.exp(_lk(z) - lt), 0.0)
    acc_ref[...] += jax.lax.dot(hsT_ref[:, sds], e,
                                preferred_element_type=jnp.float32)
    l_ref[...] += jax.lax.dot(jnp.ones((1, ts), jnp.float32), e,
                              preferred_element_type=jnp.float32)

    @pl.when(s_id == gs - 1)
    def _():
        l = l_ref[...]
        r = jnp.where(l > 0, acc_ref[...] / (l + 1e-9),
                      hsumT_ref[...] / (ns + 1e-9))
        o_ref[...] = r.T


def _col_flash(xs, ws, avs, xt, wt, avt, adj):
    ns, nt = adj.shape
    ts = min(1024, ns)
    tt = min(512, nt)
    gs, gt = ns // ts, nt // tt
    body = functools.partial(_col_body, gs, ts, tt, float(ns))
    return pl.pallas_call(
        body,
        grid=(gt, gs),
        in_specs=[
            pl.BlockSpec((ns, D), lambda t, s: (0, 0)),
            pl.BlockSpec((D, D), lambda t, s: (0, 0)),
            pl.BlockSpec((D, 1), lambda t, s: (0, 0)),
            pl.BlockSpec((nt, D), lambda t, s: (0, 0)),
            pl.BlockSpec((D, D), lambda t, s: (0, 0)),
            pl.BlockSpec((D, 1), lambda t, s: (0, 0)),
            pl.BlockSpec((ts, tt), lambda t, s: (s, t)),
        ],
        out_specs=pl.BlockSpec((tt, D), lambda t, s: (t, 0)),
        out_shape=jax.ShapeDtypeStruct((nt, D), jnp.float32),
        scratch_shapes=[
            pltpu.VMEM((D, ns), jnp.float32),
            pltpu.VMEM((ns, 1), jnp.float32),
            pltpu.VMEM((1, nt), jnp.float32),
            pltpu.VMEM((1, 1), jnp.float32),
            pltpu.VMEM((D, 1), jnp.float32),
            pltpu.VMEM((D, tt), jnp.float32),
            pltpu.VMEM((1, tt), jnp.float32),
        ],
    )(xs, ws, avs.reshape(D, 1), xt, wt, avt.reshape(D, 1), adj)


def _dual_body(gi, gj, ti, tj, ns, nt, has_extra_s, leaky_s, leaky_t, *refs):
    xs_ref, ws_ref, avs_ref, xt_ref, wt_ref, avt_ref, a_ref = refs[:7]
    idx = 7
    es_ref = None
    if has_extra_s:
        es_ref = refs[idx]
        idx += 1
    os_ref, ot_ref = refs[idx], refs[idx + 1]
    (hs_ref, qs_ref, qsT_ref, qm_ref, hssT_ref,
     ht_ref, kt_ref, km_ref, hts_ref,
     lr_ref, lc_ref, otT_ref, hsT_ref) = refs[idx + 2:]
    i = pl.program_id(0)
    j = pl.program_id(1)
    rds = pl.ds(i * ti, ti)
    cds = pl.ds(j * tj, tj)

    @pl.when((i == 0) & (j == 0))
    def _():
        hs = jax.lax.dot(xs_ref[...], ws_ref[...],
                         preferred_element_type=jnp.float32)
        hs_ref[...] = hs
        q = jax.lax.dot(hs, avs_ref[...], preferred_element_type=jnp.float32)
        qs_ref[...] = q
        qsT_ref[...] = q.T
        qm_ref[...] = jnp.max(q, keepdims=True)
        hssT_ref[...] = jnp.sum(hs, axis=0, keepdims=True).T
        ht = jax.lax.dot(xt_ref[...], wt_ref[...],
                         preferred_element_type=jnp.float32)
        ht_ref[...] = ht
        k = jax.lax.dot(ht, avt_ref[...], preferred_element_type=jnp.float32)
        kt_ref[...] = k.T
        km_ref[...] = jnp.max(k, keepdims=True)
        hts_ref[...] = jnp.sum(ht, axis=0, keepdims=True)

    km = km_ref[...]
    qm = qm_ref[...]
    lmax = _lk(qm + km)
    q = qs_ref[rds, :]
    k = kt_ref[:, cds]
    li = _lk(q + km)

    @pl.when(j == 0)
    def _():
        os_ref[...] = jnp.zeros(os_ref.shape, jnp.float32)
        lr_ref[...] = jnp.zeros(lr_ref.shape, jnp.float32)
        wt = jnp.exp(_lk(qsT_ref[:, rds] + km) - lmax)
        hsT_ref[...] = hs_ref[rds, :].T * wt

    @pl.when(i == 0)
    def _():
        otT_ref[:, cds] = jnp.zeros((D, tj), jnp.float32)
        lc_ref[:, cds] = jnp.zeros((1, tj), jnp.float32)

    z = q + k
    e = jnp.where(a_ref[...] > THRESH, jnp.exp(_lk(z) - li), 0.0)

    os_ref[...] += jax.lax.dot(e, ht_ref[cds, :],
                               preferred_element_type=jnp.float32)
    lr_ref[...] += jax.lax.dot(e, jnp.ones((tj, 1), jnp.float32),
                               preferred_element_type=jnp.float32)

    @pl.when(j == gj - 1)
    def _():
        l = lr_ref[...]
        r = jnp.where(l > 0, os_ref[...] / (l + 1e-9),
                      hts_ref[...] / (nt + 1e-9))
        if has_extra_s:
            r = r + es_ref[...]
        if leaky_s:
            r = jnp.maximum(r, SLOPE * r)
        os_ref[...] = r

    wt = jnp.exp(_lk(qsT_ref[:, rds] + km) - lmax)
    otT_ref[:, cds] += jax.lax.dot(hsT_ref[...], e,
                                   preferred_element_type=jnp.float32)
    lc_ref[:, cds] += jax.lax.dot(wt, e, preferred_element_type=jnp.float32)

    @pl.when(i == gi - 1)
    def _():
        lt = _lk(qm + k)
        f = jnp.exp(lmax - lt)
        lf = lc_ref[:, cds] * f
        rt = jnp.where(lf > 0, (otT_ref[:, cds] * f) / (lf + 1e-9),
                       hssT_ref[...] / (ns + 1e-9))
        if leaky_t:
            rt = jnp.maximum(rt, SLOPE * rt)
        ot_ref[cds, :] = rt.T


def _dual_flash(xs, ws, avs, xt, wt, avt, adj,
                extra_s=None, leaky_s=False, leaky_t=False):
    ns, nt = adj.shape
    ti = min(512, ns)
    tj = min(1024, nt)
    gi, gj = ns // ti, nt // tj
    inputs = [xs, ws, avs.reshape(D, 1), xt, wt, avt.reshape(D, 1), adj]
    specs = [
        pl.BlockSpec((ns, D), lambda i, j: (0, 0)),
        pl.BlockSpec((D, D), lambda i, j: (0, 0)),
        pl.BlockSpec((D, 1), lambda i, j: (0, 0)),
        pl.BlockSpec((nt, D), lambda i, j: (0, 0)),
        pl.BlockSpec((D, D), lambda i, j: (0, 0)),
        pl.BlockSpec((D, 1), lambda i, j: (0, 0)),
        pl.BlockSpec((ti, tj), lambda i, j: (i, j)),
    ]
    if extra_s is not None:
        inputs.append(extra_s)
        specs.append(pl.BlockSpec((ti, D), lambda i, j: (i, 0)))
    body = functools.partial(_dual_body, gi, gj, ti, tj, float(ns), float(nt),
                             extra_s is not None, leaky_s, leaky_t)
    return pl.pallas_call(
        body,
        grid=(gi, gj),
        in_specs=specs,
        out_specs=[
            pl.BlockSpec((ti, D), lambda i, j: (i, 0)),
            pl.BlockSpec((nt, D), lambda i, j: (0, 0)),
        ],
        out_shape=[
            jax.ShapeDtypeStruct((ns, D), jnp.float32),
            jax.ShapeDtypeStruct((nt, D), jnp.float32),
        ],
        scratch_shapes=[
            pltpu.VMEM((ns, D), jnp.float32),
            pltpu.VMEM((ns, 1), jnp.float32),
            pltpu.VMEM((1, ns), jnp.float32),
            pltpu.VMEM((1, 1), jnp.float32),
            pltpu.VMEM((D, 1), jnp.float32),
            pltpu.VMEM((nt, D), jnp.float32),
            pltpu.VMEM((1, nt), jnp.float32),
            pltpu.VMEM((1, 1), jnp.float32),
            pltpu.VMEM((1, D), jnp.float32),
            pltpu.VMEM((ti, 1), jnp.float32),
            pltpu.VMEM((1, nt), jnp.float32),
            pltpu.VMEM((D, nt), jnp.float32),
            pltpu.VMEM((D, ti), jnp.float32),
        ],
    )(*inputs)


def kernel(x_0, x_1, x_2, a0, a1, coa2, b1, b2, params):
    p = params

    def dec_params(pre):
        return (p[pre + 'W_in'], p[pre + 'b_in'].reshape(1, H),
                p[pre + 'W_mid'], p[pre + 'b_mid'].reshape(1, H),
                p[pre + 'W_out'], p[pre + 'b_out'].reshape(1, D))

    s0, t1 = _dual_flash(x_0, p['Ws_b1_1'], p['as_b1_1'],
                         x_1, p['Wt_b1_1'], p['at_b1_1'], b1)
    x1_l1, x2_l1 = _dual_flash(x_1, p['Ws_b2_1'], p['as_b2_1'],
                               x_2, p['Wt_b2_1'], p['at_b2_1'], b2,
                               extra_s=t1, leaky_s=True, leaky_t=True)
    x0_l1 = _row_flash(x_0, p['W_a0_1'], p['aq_a0_1'], p['ak_a0_1'], a0,
                       extra=s0, leaky_out=True)

    t1_2 = _col_flash(x0_l1, p['Ws_b1_2'], p['as_b1_2'],
                      x1_l1, p['Wt_b1_2'], p['at_b1_2'], b1)
    t2_2 = _col_flash(x1_l1, p['Ws_b2_2'], p['as_b2_2'],
                      x2_l1, p['Wt_b2_2'], p['at_b2_2'], b2)

    out0 = _row_flash(x0_l1, p['W_a0_2'], p['aq_a0_2'], p['ak_a0_2'], a0,
                      residual=x_0, leaky_out=True, dec=dec_params('d0_'))
    out1 = _row_flash(x1_l1, p['W_a1_2'], p['aq_a1_2'], p['ak_a1_2'], a1,
                      extra=t1_2, residual=x_1, leaky_out=True,
                      dec=dec_params('d1_'))
    out2 = _row_flash(x2_l1, p['W_coa2_2'], p['aq_coa2_2'], p['ak_coa2_2'],
                      coa2, extra=t2_2, residual=x_2, leaky_out=True,
                      dec=dec_params('d2_'))
    return (out0, out1, out2)
